# baseline (speedup 1.0000x reference)
import functools

import jax
import jax.numpy as jnp
from jax import lax
from jax.experimental import pallas as pl
from jax.experimental.pallas import tpu as pltpu
from jax.experimental.pallas import tpu_sc as plsc

_B, _N, _D, _C, _K = 128, 900, 256, 81, 300
_NW = 32
_BPW = 4
_KP = 304
_NP = 1024
_R = 8


def _topk_body(logits_ref, scoresk_ref, idxk_ref, scores_ref, flat_ref,
               lflat_ref):
    step = pl.program_id(0)
    pos = (lax.broadcasted_iota(jnp.int32, (8, 128), 0) * 128
           + lax.broadcasted_iota(jnp.int32, (8, 128), 1))
    for r in range(_R):
        x = logits_ref[r]
        lane = lax.broadcasted_iota(jnp.int32, (_N, _C), 1)
        x = jnp.where(lane < _C - 1, x, -jnp.inf)
        m2 = jnp.max(x, axis=1, keepdims=True)
        mp = jnp.concatenate(
            [m2, jnp.full((_NP - _N, 1), -jnp.inf, jnp.float32)], axis=0)
        z = mp.reshape(8, 128)
        u = lax.bitcast_convert_type(z, jnp.uint32)
        key = jnp.where((u >> 31) != 0, ~u, u | jnp.uint32(0x80000000))
        idx = pos

        for kk_log in range(1, 11):
            kk = 1 << kk_log
            gf = (pos & kk) == 0
            for d_log in range(kk_log - 1, -1, -1):
                d = 1 << d_log
                bit = (pos & d) != 0
                if d < 128:
                    rk_p, rk_m = pltpu.roll(key, d, 1), pltpu.roll(key, 128 - d, 1)
                    ri_p, ri_m = pltpu.roll(idx, d, 1), pltpu.roll(idx, 128 - d, 1)
                else:
                    sd = d // 128
                    rk_p, rk_m = pltpu.roll(key, sd, 0), pltpu.roll(key, 8 - sd, 0)
                    ri_p, ri_m = pltpu.roll(idx, sd, 0), pltpu.roll(idx, 8 - sd, 0)
                pk = jnp.where(bit, rk_p, rk_m)
                pi = jnp.where(bit, ri_p, ri_m)
                p_first = (pk > key) | ((pk == key) & (pi < idx))
                take = ~(bit ^ gf ^ p_first)
                key = jnp.where(take, pk, key)
                idx = jnp.where(take, pi, idx)

        um = jnp.where((key >> 31) != 0, key & jnp.uint32(0x7FFFFFFF), ~key)
        ms = lax.bitcast_convert_type(um, jnp.float32)
        sig = jax.nn.sigmoid(ms)
        scores_ref[r] = sig
        scoresk_ref[r] = sig.reshape(1, _NP)[0, :_K]
        idxk_ref[r] = idx.reshape(1, _NP)[0, :_K]
        b = step * _R + r
        safe = jnp.minimum(idx, _N - 1)
        flat_ref[r] = safe + b * _N
        lflat_ref[r] = safe + (b % _BPW) * _N


def _run_topk(lesion_logits):
    nsteps = _B // _R
    return pl.pallas_call(
        _topk_body,
        grid=(nsteps,),
        in_specs=[pl.BlockSpec((_R, _N, _C), lambda i: (i, 0, 0))],
        out_specs=[
            pl.BlockSpec((_R, _K), lambda i: (i, 0)),
            pl.BlockSpec((_R, _K), lambda i: (i, 0)),
            pl.BlockSpec((_R, 8, 128), lambda i: (i, 0, 0)),
            pl.BlockSpec((_R, 8, 128), lambda i: (i, 0, 0)),
            pl.BlockSpec((_R, 8, 128), lambda i: (i, 0, 0)),
        ],
        out_shape=[
            jax.ShapeDtypeStruct((_B, _K), jnp.float32),
            jax.ShapeDtypeStruct((_B, _K), jnp.int32),
            jax.ShapeDtypeStruct((_B, 8, 128), jnp.float32),
            jax.ShapeDtypeStruct((_B, 8, 128), jnp.int32),
            jax.ShapeDtypeStruct((_B, 8, 128), jnp.int32),
        ],
    )(lesion_logits)


_SRC_PER_W = _BPW * _N
_HCH = (112, 112, 80)


def _sc_gather(hidden_flat, boxes_flat, scoresf, flatf, lflatf, zeros_pad):
    mesh = plsc.VectorSubcoreMesh(core_axis_name="c", subcore_axis_name="s")

    @functools.partial(
        pl.kernel,
        out_type=[
            jax.ShapeDtypeStruct((_B * _KP, _D), jnp.float32),
            jax.ShapeDtypeStruct((_B * _KP, 128), jnp.float32),
        ],
        mesh=mesh,
        compiler_params=pltpu.CompilerParams(needs_layout_passes=False),
        scratch_types=[
            pltpu.VMEM((_KP,), jnp.int32),
            pltpu.VMEM((_KP,), jnp.int32),
            pltpu.VMEM((_KP,), jnp.float32),
            pltpu.VMEM((max(_HCH), _D), jnp.float32),
            pltpu.VMEM((_SRC_PER_W * 4,), jnp.float32),
            pltpu.VMEM((_KP, 128), jnp.float32),
            pltpu.SemaphoreType.DMA,
        ],
    )
    def k(hid_hbm, box_hbm, sc_hbm, idx_hbm, lidx_hbm, zero_hbm,
          out_hbm, aug_hbm, idx_v, lidx_v, sc_v, rows_v, boxsrc_v, aug_v, sem):
        nc = 2
        wid = lax.axis_index("s") * nc + lax.axis_index("c")
        pltpu.sync_copy(box_hbm.at[pl.ds(wid * _SRC_PER_W * 4, _SRC_PER_W * 4)],
                        boxsrc_v)
        pltpu.sync_copy(zero_hbm, aug_v)
        lane = lax.iota(jnp.int32, 16)
        for be in range(_BPW):
            b = wid * _BPW + be
            srt = b * _NP
            obase = b * _KP
            pltpu.sync_copy(idx_hbm.at[pl.ds(srt, _KP)], idx_v)
            pltpu.sync_copy(lidx_hbm.at[pl.ds(srt, _KP)], lidx_v)
            pltpu.sync_copy(sc_hbm.at[pl.ds(srt, _KP)], sc_v)
            off = 0
            for n in _HCH:
                idx_c = idx_v.at[pl.ds(off, n)]
                dst = rows_v.at[pl.ds(0, n)]
                pltpu.async_copy(hid_hbm.at[idx_c], dst, sem).wait()
                pltpu.sync_copy(dst, out_hbm.at[pl.ds(obase + off, n)])
                off += n
            for g in range(_KP // 16):
                l16 = lidx_v[pl.ds(g * 16, 16)] * 4
                p16 = lane + g * 16
                for col in range(4):
                    vals = plsc.load_gather(boxsrc_v, [l16 + col])
                    plsc.store_scatter(
                        aug_v, [p16, jnp.full((16,), col, jnp.int32)], vals)
                s16 = sc_v[pl.ds(g * 16, 16)]
                plsc.store_scatter(
                    aug_v, [p16, jnp.full((16,), 4, jnp.int32)], s16)
            pltpu.sync_copy(aug_v, aug_hbm.at[pl.ds(obase, _KP)])

    return k(hidden_flat, boxes_flat, scoresf, flatf, lflatf, zeros_pad)


def _gelu_exact(x):
    return x * 0.5 * (1.0 + lax.erf(x * 0.7071067811865476))


def _mlp_body(hid_ref, aug_ref, w1_ref, b1_ref, wcat_ref, bsum_ref,
              g_ref, beta_ref, out_ref):
    a = aug_ref[0]
    h = _gelu_exact(jnp.dot(a, w1_ref[...],
                            preferred_element_type=jnp.float32) + b1_ref[...])
    t = jnp.dot(h.astype(jnp.bfloat16), wcat_ref[...],
                preferred_element_type=jnp.float32)
    tok = hid_ref[0] + t + bsum_ref[...]
    mu = jnp.mean(tok, axis=1, keepdims=True)
    var = jnp.mean((tok - mu) ** 2, axis=1, keepdims=True)
    res = (tok - mu) * lax.rsqrt(var + 1e-5) * g_ref[...] + beta_ref[...]
    out_ref[0] = res[:_K, :]


def _run_mlp(hid, aug, w1cat, b1cat, wcat, bsum, ln_g, ln_b):
    blk = lambda i: (i, 0, 0)
    rep = lambda i: (0, 0)
    return pl.pallas_call(
        _mlp_body,
        grid=(_B,),
        in_specs=[
            pl.BlockSpec((1, _KP, _D), blk),
            pl.BlockSpec((1, _KP, 128), blk),
            pl.BlockSpec((128, 2 * _D), rep),
            pl.BlockSpec((1, 2 * _D), rep),
            pl.BlockSpec((2 * _D, _D), rep),
            pl.BlockSpec((1, _D), rep),
            pl.BlockSpec((1, _D), rep),
            pl.BlockSpec((1, _D), rep),
        ],
        out_specs=pl.BlockSpec((1, _K, _D), blk),
        out_shape=jax.ShapeDtypeStruct((_B, _K, _D), jnp.float32),
    )(hid, aug, w1cat, b1cat, wcat, bsum, ln_g, ln_b)


def kernel(lesion_hidden, lesion_boxes, lesion_logits, box_W1, box_b1,
           box_W2, box_b2, sc_W1, sc_b1, sc_W2, sc_b2, ln_g, ln_b):
    topk_scores, topk_indices, scores_s, flat_s, lflat_s = _run_topk(
        lesion_logits)

    hidden_flat = lesion_hidden.reshape(_B * _N, _D)
    boxes_flat = lesion_boxes.reshape(_B * _N * 4)
    zeros_pad = jnp.zeros((_KP, 128), jnp.float32)
    hidg, aug = _sc_gather(
        hidden_flat, boxes_flat, scores_s.reshape(_B * _NP),
        flat_s.reshape(_B * _NP), lflat_s.reshape(_B * _NP), zeros_pad)

    w1cat = jnp.zeros((128, 2 * _D), jnp.float32)
    w1cat = w1cat.at[0:4, 0:_D].set(box_W1).at[4, _D:].set(sc_W1[0])
    b1cat = jnp.concatenate([box_b1, sc_b1]).reshape(1, 2 * _D)
    wcat = jnp.concatenate([box_W2, sc_W2], axis=0).astype(jnp.bfloat16)
    bsum = (box_b2 + sc_b2).reshape(1, _D)
    out = _run_mlp(
        hidg.reshape(_B, _KP, _D), aug.reshape(_B, _KP, 128),
        w1cat, b1cat, wcat, bsum, ln_g.reshape(1, _D), ln_b.reshape(1, _D))
    return (out, topk_scores, topk_indices)


# speedup vs baseline: 1.1011x; 1.1011x over previous
import functools

import jax
import jax.numpy as jnp
from jax import lax
from jax.experimental import pallas as pl
from jax.experimental.pallas import tpu as pltpu
from jax.experimental.pallas import tpu_sc as plsc

_B, _N, _D, _C, _K = 128, 900, 256, 81, 300
_NW = 32
_BPW = 4
_KP = 304
_NP = 1024
_R = 8


def _topk_body(logits_ref, scoresk_ref, idxk_ref, sc3_ref, id3_ref,
               flat3_ref):
    step = pl.program_id(0)
    pos = (lax.broadcasted_iota(jnp.int32, (8, 128), 0) * 128
           + lax.broadcasted_iota(jnp.int32, (8, 128), 1))
    for r in range(_R):
        x = logits_ref[r]
        lane = lax.broadcasted_iota(jnp.int32, (_N, _C), 1)
        x = jnp.where(lane < _C - 1, x, -jnp.inf)
        m2 = jnp.max(x, axis=1, keepdims=True)
        mp = jnp.concatenate(
            [m2, jnp.full((_NP - _N, 1), -jnp.inf, jnp.float32)], axis=0)
        z = mp.reshape(8, 128)
        u = lax.bitcast_convert_type(z, jnp.uint32)
        key = jnp.where((u >> 31) != 0, ~u, u | jnp.uint32(0x80000000))
        idx = pos

        for kk_log in range(1, 11):
            kk = 1 << kk_log
            gf = (pos & kk) == 0
            for d_log in range(kk_log - 1, -1, -1):
                d = 1 << d_log
                bit = (pos & d) != 0
                if d < 128:
                    rk_p, rk_m = pltpu.roll(key, d, 1), pltpu.roll(key, 128 - d, 1)
                    ri_p, ri_m = pltpu.roll(idx, d, 1), pltpu.roll(idx, 128 - d, 1)
                else:
                    sd = d // 128
                    rk_p, rk_m = pltpu.roll(key, sd, 0), pltpu.roll(key, 8 - sd, 0)
                    ri_p, ri_m = pltpu.roll(idx, sd, 0), pltpu.roll(idx, 8 - sd, 0)
                pk = jnp.where(bit, rk_p, rk_m)
                pi = jnp.where(bit, ri_p, ri_m)
                p_first = (pk > key) | ((pk == key) & (pi < idx))
                take = ~(bit ^ gf ^ p_first)
                key = jnp.where(take, pk, key)
                idx = jnp.where(take, pi, idx)

        um = jnp.where((key >> 31) != 0, key & jnp.uint32(0x7FFFFFFF), ~key)
        ms = lax.bitcast_convert_type(um, jnp.float32)
        sig = jax.nn.sigmoid(ms)
        scoresk_ref[r] = sig.reshape(1, _NP)[0, :_K]
        idxk_ref[r] = idx.reshape(1, _NP)[0, :_K]
        sc3_ref[r] = sig
        id3_ref[r] = idx
        b = step * _R + r
        safe = jnp.minimum(idx, _N - 1)
        flat3_ref[r] = (safe + b * _N)[0:3, :]


def _run_topk(lesion_logits):
    nsteps = _B // _R
    return pl.pallas_call(
        _topk_body,
        grid=(nsteps,),
        in_specs=[pl.BlockSpec((_R, _N, _C), lambda i: (i, 0, 0))],
        out_specs=[
            pl.BlockSpec((_R, _K), lambda i: (i, 0)),
            pl.BlockSpec((_R, _K), lambda i: (i, 0)),
            pl.BlockSpec((_R, 8, 128), lambda i: (i, 0, 0)),
            pl.BlockSpec((_R, 8, 128), lambda i: (i, 0, 0)),
            pl.BlockSpec((_R, 3, 128), lambda i: (i, 0, 0)),
        ],
        out_shape=[
            jax.ShapeDtypeStruct((_B, _K), jnp.float32),
            jax.ShapeDtypeStruct((_B, _K), jnp.int32),
            jax.ShapeDtypeStruct((_B, 8, 128), jnp.float32),
            jax.ShapeDtypeStruct((_B, 8, 128), jnp.int32),
            jax.ShapeDtypeStruct((_B, 3, 128), jnp.int32),
        ],
    )(lesion_logits)


_HCH = (112, 112, 80)


def _sc_gather(hidden_flat, flatf):
    mesh = plsc.VectorSubcoreMesh(core_axis_name="c", subcore_axis_name="s")

    @functools.partial(
        pl.kernel,
        out_type=jax.ShapeDtypeStruct((_B * _KP, _D), jnp.float32),
        mesh=mesh,
        compiler_params=pltpu.CompilerParams(needs_layout_passes=False),
        scratch_types=[
            pltpu.VMEM((_KP,), jnp.int32),
            pltpu.VMEM((max(_HCH), _D), jnp.float32),
            pltpu.SemaphoreType.DMA,
        ],
    )
    def k(hid_hbm, idx_hbm, out_hbm, idx_v, rows_v, sem):
        nc = 2
        wid = lax.axis_index("s") * nc + lax.axis_index("c")
        for be in range(_BPW):
            b = wid * _BPW + be
            pltpu.sync_copy(idx_hbm.at[pl.ds(b * 384, _KP)], idx_v)
            obase = b * _KP
            off = 0
            for n in _HCH:
                idx_c = idx_v.at[pl.ds(off, n)]
                dst = rows_v.at[pl.ds(0, n)]
                pltpu.async_copy(hid_hbm.at[idx_c], dst, sem).wait()
                pltpu.sync_copy(dst, out_hbm.at[pl.ds(obase + off, n)])
                off += n

    return k(hidden_flat, flatf)


def _gelu_exact(x):
    return x * 0.5 * (1.0 + lax.erf(x * 0.7071067811865476))


def _mlp_body(hid_ref, box_ref, sc3_ref, id3_ref, w1_ref, b1_ref,
              scw1_ref, scb1_ref, wcat_ref, bsum_ref, g_ref, beta_ref,
              out_ref):
    idT = jnp.transpose(id3_ref[0])
    scT = jnp.transpose(sc3_ref[0])
    icol = jnp.concatenate(
        [idT[:, 0:1], idT[:, 1:2], idT[:, 2:3]], axis=0)[:_KP]
    scol = jnp.concatenate(
        [scT[:, 0:1], scT[:, 1:2], scT[:, 2:3]], axis=0)[:_KP]
    lanes = lax.broadcasted_iota(jnp.int32, (_KP, _N), 1)
    oh = (lanes == icol).astype(jnp.float32)
    boxg = jnp.dot(oh, box_ref[0], preferred_element_type=jnp.float32)
    h_box = _gelu_exact(jnp.dot(boxg, w1_ref[...],
                                preferred_element_type=jnp.float32)
                        + b1_ref[...])
    h_sc = _gelu_exact(scol * scw1_ref[...] + scb1_ref[...])
    hcat = jnp.concatenate([h_box, h_sc], axis=1).astype(jnp.bfloat16)
    t = jnp.dot(hcat, wcat_ref[...], preferred_element_type=jnp.float32)
    tok = hid_ref[0] + t + bsum_ref[...]
    mu = jnp.mean(tok, axis=1, keepdims=True)
    var = jnp.mean((tok - mu) ** 2, axis=1, keepdims=True)
    res = (tok - mu) * lax.rsqrt(var + 1e-5) * g_ref[...] + beta_ref[...]
    out_ref[0] = res[:_K, :]


def _run_mlp(hid, boxes, sc3, id3, box_W1, box_b1, sc_W1, sc_b1, wcat, bsum,
             ln_g, ln_b):
    blk = lambda i: (i, 0, 0)
    rep = lambda i: (0, 0)
    return pl.pallas_call(
        _mlp_body,
        grid=(_B,),
        in_specs=[
            pl.BlockSpec((1, _KP, _D), blk),
            pl.BlockSpec((1, _N, 4), blk),
            pl.BlockSpec((1, 8, 128), blk),
            pl.BlockSpec((1, 8, 128), blk),
            pl.BlockSpec((4, _D), rep),
            pl.BlockSpec((1, _D), rep),
            pl.BlockSpec((1, _D), rep),
            pl.BlockSpec((1, _D), rep),
            pl.BlockSpec((2 * _D, _D), rep),
            pl.BlockSpec((1, _D), rep),
            pl.BlockSpec((1, _D), rep),
            pl.BlockSpec((1, _D), rep),
        ],
        out_specs=pl.BlockSpec((1, _K, _D), blk),
        out_shape=jax.ShapeDtypeStruct((_B, _K, _D), jnp.float32),
    )(hid, boxes, sc3, id3, box_W1, box_b1, sc_W1, sc_b1, wcat, bsum,
      ln_g, ln_b)


def kernel(lesion_hidden, lesion_boxes, lesion_logits, box_W1, box_b1,
           box_W2, box_b2, sc_W1, sc_b1, sc_W2, sc_b2, ln_g, ln_b):
    topk_scores, topk_indices, sc3, id3, flat3 = _run_topk(lesion_logits)

    hidden_flat = lesion_hidden.reshape(_B * _N, _D)
    hidg = _sc_gather(hidden_flat, flat3.reshape(_B * 384))

    wcat = jnp.concatenate([box_W2, sc_W2], axis=0).astype(jnp.bfloat16)
    bsum = (box_b2 + sc_b2).reshape(1, _D)
    out = _run_mlp(
        hidg.reshape(_B, _KP, _D), lesion_boxes, sc3, id3,
        box_W1, box_b1.reshape(1, _D), sc_W1, sc_b1.reshape(1, _D),
        wcat, bsum, ln_g.reshape(1, _D), ln_b.reshape(1, _D))
    return (out, topk_scores, topk_indices)


# speedup vs baseline: 1.1496x; 1.0440x over previous
import functools

import jax
import jax.numpy as jnp
from jax import lax
from jax.experimental import pallas as pl
from jax.experimental.pallas import tpu as pltpu
from jax.experimental.pallas import tpu_sc as plsc

_B, _N, _D, _C, _K = 128, 900, 256, 81, 300
_NW = 32
_BPW = 4
_KP = 304
_NP = 1024
_R = 8


def _topk_body(logits_ref, scoresk_ref, idxk_ref, sc3_ref, id3_ref,
               flat3_ref):
    step = pl.program_id(0)
    pos = (lax.broadcasted_iota(jnp.int32, (8, 128), 0) * 128
           + lax.broadcasted_iota(jnp.int32, (8, 128), 1))
    for r in range(_R):
        x = logits_ref[r]
        lane = lax.broadcasted_iota(jnp.int32, (_N, _C), 1)
        x = jnp.where(lane < _C - 1, x, -jnp.inf)
        m2 = jnp.max(x, axis=1, keepdims=True)
        mp = jnp.concatenate(
            [m2, jnp.full((_NP - _N, 1), -jnp.inf, jnp.float32)], axis=0)
        z = mp.reshape(8, 128)
        u = lax.bitcast_convert_type(z, jnp.uint32)
        key = jnp.where((u >> 31) != 0, ~u, u | jnp.uint32(0x80000000))
        idx = pos

        for kk_log in range(1, 11):
            kk = 1 << kk_log
            gf = (pos & kk) == 0
            for d_log in range(kk_log - 1, -1, -1):
                d = 1 << d_log
                bit = (pos & d) != 0
                if d < 128:
                    rk_p, rk_m = pltpu.roll(key, d, 1), pltpu.roll(key, 128 - d, 1)
                    ri_p, ri_m = pltpu.roll(idx, d, 1), pltpu.roll(idx, 128 - d, 1)
                else:
                    sd = d // 128
                    rk_p, rk_m = pltpu.roll(key, sd, 0), pltpu.roll(key, 8 - sd, 0)
                    ri_p, ri_m = pltpu.roll(idx, sd, 0), pltpu.roll(idx, 8 - sd, 0)
                pk = jnp.where(bit, rk_p, rk_m)
                pi = jnp.where(bit, ri_p, ri_m)
                p_first = (pk > key) | ((pk == key) & (pi < idx))
                take = ~(bit ^ gf ^ p_first)
                key = jnp.where(take, pk, key)
                idx = jnp.where(take, pi, idx)

        um = jnp.where((key >> 31) != 0, key & jnp.uint32(0x7FFFFFFF), ~key)
        ms = lax.bitcast_convert_type(um, jnp.float32)
        sig = jax.nn.sigmoid(ms)
        scoresk_ref[r] = sig.reshape(1, _NP)[0, :_K]
        idxk_ref[r] = idx.reshape(1, _NP)[0, :_K]
        sc3_ref[r] = sig
        id3_ref[r] = idx
        b = step * _R + r
        safe = jnp.minimum(idx, _N - 1)
        flat3_ref[r] = safe[0:3, :]


def _run_topk(lesion_logits):
    nsteps = _B // _R
    return pl.pallas_call(
        _topk_body,
        grid=(nsteps,),
        in_specs=[pl.BlockSpec((_R, _N, _C), lambda i: (i, 0, 0))],
        out_specs=[
            pl.BlockSpec((_R, _K), lambda i: (i, 0)),
            pl.BlockSpec((_R, _K), lambda i: (i, 0)),
            pl.BlockSpec((_R, 8, 128), lambda i: (i, 0, 0)),
            pl.BlockSpec((_R, 8, 128), lambda i: (i, 0, 0)),
            pl.BlockSpec((_R, 3, 128), lambda i: (i, 0, 0)),
        ],
        out_shape=[
            jax.ShapeDtypeStruct((_B, _K), jnp.float32),
            jax.ShapeDtypeStruct((_B, _K), jnp.int32),
            jax.ShapeDtypeStruct((_B, 8, 128), jnp.float32),
            jax.ShapeDtypeStruct((_B, 8, 128), jnp.int32),
            jax.ShapeDtypeStruct((_B, 3, 128), jnp.int32),
        ],
    )(lesion_logits)


_HCH = (112, 112, 80)


def _sc_gather(hidden, flatf):
    mesh = plsc.VectorSubcoreMesh(core_axis_name="c", subcore_axis_name="s")

    @functools.partial(
        pl.kernel,
        out_type=jax.ShapeDtypeStruct((_B * _KP, _D), jnp.float32),
        mesh=mesh,
        compiler_params=pltpu.CompilerParams(needs_layout_passes=False),
        scratch_types=[
            pltpu.VMEM((_KP,), jnp.int32),
            pltpu.VMEM((max(_HCH), _D), jnp.float32),
            pltpu.SemaphoreType.DMA,
        ],
    )
    def k(hid_hbm, idx_hbm, out_hbm, idx_v, rows_v, sem):
        nc = 2
        wid = lax.axis_index("s") * nc + lax.axis_index("c")
        for be in range(_BPW):
            b = wid * _BPW + be
            pltpu.sync_copy(idx_hbm.at[pl.ds(b * 384, _KP)], idx_v)
            slab = hid_hbm.at[b]
            obase = b * _KP
            off = 0
            for n in _HCH:
                idx_c = idx_v.at[pl.ds(off, n)]
                dst = rows_v.at[pl.ds(0, n)]
                pltpu.async_copy(slab.at[idx_c], dst, sem).wait()
                pltpu.sync_copy(dst, out_hbm.at[pl.ds(obase + off, n)])
                off += n

    return k(hidden, flatf)


def _gelu_exact(x):
    return x * 0.5 * (1.0 + lax.erf(x * 0.7071067811865476))


def _mlp_body(hid_ref, box_ref, sc3_ref, id3_ref, w1_ref, b1_ref,
              scw1_ref, scb1_ref, wcat_ref, bsum_ref, g_ref, beta_ref,
              out_ref):
    idT = jnp.transpose(id3_ref[0])
    scT = jnp.transpose(sc3_ref[0])
    icol = jnp.concatenate(
        [idT[:, 0:1], idT[:, 1:2], idT[:, 2:3]], axis=0)[:_KP]
    scol = jnp.concatenate(
        [scT[:, 0:1], scT[:, 1:2], scT[:, 2:3]], axis=0)[:_KP]
    lanes = lax.broadcasted_iota(jnp.int32, (_KP, _N), 1)
    oh = (lanes == icol).astype(jnp.float32)
    boxg = jnp.dot(oh, box_ref[0], preferred_element_type=jnp.float32)
    h_box = _gelu_exact(jnp.dot(boxg, w1_ref[...],
                                preferred_element_type=jnp.float32)
                        + b1_ref[...])
    h_sc = _gelu_exact(scol * scw1_ref[...] + scb1_ref[...])
    hcat = jnp.concatenate([h_box, h_sc], axis=1).astype(jnp.bfloat16)
    t = jnp.dot(hcat, wcat_ref[...], preferred_element_type=jnp.float32)
    tok = hid_ref[0] + t + bsum_ref[...]
    mu = jnp.mean(tok, axis=1, keepdims=True)
    var = jnp.mean((tok - mu) ** 2, axis=1, keepdims=True)
    res = (tok - mu) * lax.rsqrt(var + 1e-5) * g_ref[...] + beta_ref[...]
    out_ref[0] = res[:_K, :]


def _run_mlp(hid, boxes, sc3, id3, box_W1, box_b1, sc_W1, sc_b1, wcat, bsum,
             ln_g, ln_b):
    blk = lambda i: (i, 0, 0)
    rep = lambda i: (0, 0)
    return pl.pallas_call(
        _mlp_body,
        grid=(_B,),
        in_specs=[
            pl.BlockSpec((1, _KP, _D), blk),
            pl.BlockSpec((1, _N, 4), blk),
            pl.BlockSpec((1, 8, 128), blk),
            pl.BlockSpec((1, 8, 128), blk),
            pl.BlockSpec((4, _D), rep),
            pl.BlockSpec((1, _D), rep),
            pl.BlockSpec((1, _D), rep),
            pl.BlockSpec((1, _D), rep),
            pl.BlockSpec((2 * _D, _D), rep),
            pl.BlockSpec((1, _D), rep),
            pl.BlockSpec((1, _D), rep),
            pl.BlockSpec((1, _D), rep),
        ],
        out_specs=pl.BlockSpec((1, _K, _D), blk),
        out_shape=jax.ShapeDtypeStruct((_B, _K, _D), jnp.float32),
    )(hid, boxes, sc3, id3, box_W1, box_b1, sc_W1, sc_b1, wcat, bsum,
      ln_g, ln_b)


def kernel(lesion_hidden, lesion_boxes, lesion_logits, box_W1, box_b1,
           box_W2, box_b2, sc_W1, sc_b1, sc_W2, sc_b2, ln_g, ln_b):
    topk_scores, topk_indices, sc3, id3, flat3 = _run_topk(lesion_logits)

    hidg = _sc_gather(lesion_hidden, flat3.reshape(_B * 384))

    wcat = jnp.concatenate([box_W2, sc_W2], axis=0).astype(jnp.bfloat16)
    bsum = (box_b2 + sc_b2).reshape(1, _D)
    out = _run_mlp(
        hidg.reshape(_B, _KP, _D), lesion_boxes, sc3, id3,
        box_W1, box_b1.reshape(1, _D), sc_W1, sc_b1.reshape(1, _D),
        wcat, bsum, ln_g.reshape(1, _D), ln_b.reshape(1, _D))
    return (out, topk_scores, topk_indices)


# speedup vs baseline: 1.3840x; 1.2039x over previous
import functools

import jax
import jax.numpy as jnp
from jax import lax
from jax.experimental import pallas as pl
from jax.experimental.pallas import tpu as pltpu
from jax.experimental.pallas import tpu_sc as plsc

_B, _N, _D, _C, _K = 128, 900, 256, 81, 300
_NW = 32
_BPW = 4
_KP = 304
_NP = 1024
_R = 8


def _topk_body(logits_ref, scoresk_ref, idxk_ref, sc3_ref, id3_ref,
               flat3_ref):
    step = pl.program_id(0)
    pos = (lax.broadcasted_iota(jnp.int32, (8, 128), 0) * 128
           + lax.broadcasted_iota(jnp.int32, (8, 128), 1))
    for r in range(_R):
        x = logits_ref[r]
        lane = lax.broadcasted_iota(jnp.int32, (_N, _C), 1)
        x = jnp.where(lane < _C - 1, x, -jnp.inf)
        m2 = jnp.max(x, axis=1, keepdims=True)
        mp = jnp.concatenate(
            [m2, jnp.full((_NP - _N, 1), -jnp.inf, jnp.float32)], axis=0)
        z = mp.reshape(8, 128)
        u = lax.bitcast_convert_type(z, jnp.uint32)
        key = jnp.where((u >> 31) != 0, ~u, u | jnp.uint32(0x80000000))
        idx = pos

        for kk_log in range(1, 11):
            kk = 1 << kk_log
            gf = (pos & kk) == 0
            for d_log in range(kk_log - 1, -1, -1):
                d = 1 << d_log
                bit = (pos & d) != 0
                if d < 128:
                    rk_p, rk_m = pltpu.roll(key, d, 1), pltpu.roll(key, 128 - d, 1)
                    ri_p, ri_m = pltpu.roll(idx, d, 1), pltpu.roll(idx, 128 - d, 1)
                else:
                    sd = d // 128
                    rk_p, rk_m = pltpu.roll(key, sd, 0), pltpu.roll(key, 8 - sd, 0)
                    ri_p, ri_m = pltpu.roll(idx, sd, 0), pltpu.roll(idx, 8 - sd, 0)
                pk = jnp.where(bit, rk_p, rk_m)
                pi = jnp.where(bit, ri_p, ri_m)
                p_first = (pk > key) | ((pk == key) & (pi < idx))
                take = ~(bit ^ gf ^ p_first)
                key = jnp.where(take, pk, key)
                idx = jnp.where(take, pi, idx)

        um = jnp.where((key >> 31) != 0, key & jnp.uint32(0x7FFFFFFF), ~key)
        ms = lax.bitcast_convert_type(um, jnp.float32)
        sig = jax.nn.sigmoid(ms)
        scoresk_ref[r] = sig.reshape(1, _NP)[0, :_K]
        idxk_ref[r] = idx.reshape(1, _NP)[0, :_K]
        sc3_ref[r] = sig
        id3_ref[r] = idx
        b = step * _R + r
        safe = jnp.minimum(idx, _N - 1)
        flat3_ref[r] = (safe * _B + b)[0:3, :]


def _run_topk(lesion_logits):
    nsteps = _B // _R
    return pl.pallas_call(
        _topk_body,
        grid=(nsteps,),
        in_specs=[pl.BlockSpec((_R, _N, _C), lambda i: (i, 0, 0))],
        out_specs=[
            pl.BlockSpec((_R, _K), lambda i: (i, 0)),
            pl.BlockSpec((_R, _K), lambda i: (i, 0)),
            pl.BlockSpec((_R, 8, 128), lambda i: (i, 0, 0)),
            pl.BlockSpec((_R, 8, 128), lambda i: (i, 0, 0)),
            pl.BlockSpec((_R, 3, 128), lambda i: (i, 0, 0)),
        ],
        out_shape=[
            jax.ShapeDtypeStruct((_B, _K), jnp.float32),
            jax.ShapeDtypeStruct((_B, _K), jnp.int32),
            jax.ShapeDtypeStruct((_B, 8, 128), jnp.float32),
            jax.ShapeDtypeStruct((_B, 8, 128), jnp.int32),
            jax.ShapeDtypeStruct((_B, 3, 128), jnp.int32),
        ],
    )(lesion_logits)


_HCH = (112, 112, 80)


def _sc_gather(hidden_t, flatf):
    mesh = plsc.VectorSubcoreMesh(core_axis_name="c", subcore_axis_name="s")

    @functools.partial(
        pl.kernel,
        out_type=jax.ShapeDtypeStruct((_B * _KP, _D), jnp.float32),
        mesh=mesh,
        compiler_params=pltpu.CompilerParams(needs_layout_passes=False),
        scratch_types=[
            pltpu.VMEM((_KP,), jnp.int32),
            pltpu.VMEM((max(_HCH), _D), jnp.float32),
            pltpu.SemaphoreType.DMA,
        ],
    )
    def k(hid_hbm, idx_hbm, out_hbm, idx_v, rows_v, sem):
        nc = 2
        wid = lax.axis_index("s") * nc + lax.axis_index("c")
        for be in range(_BPW):
            b = wid * _BPW + be
            pltpu.sync_copy(idx_hbm.at[pl.ds(b * 384, _KP)], idx_v)
            obase = b * _KP
            off = 0
            for n in _HCH:
                idx_c = idx_v.at[pl.ds(off, n)]
                dst = rows_v.at[pl.ds(0, n)]
                pltpu.async_copy(hid_hbm.at[idx_c], dst, sem).wait()
                pltpu.sync_copy(dst, out_hbm.at[pl.ds(obase + off, n)])
                off += n

    return k(hidden_t, flatf)


def _gelu_exact(x):
    return x * 0.5 * (1.0 + lax.erf(x * 0.7071067811865476))


def _mlp_body(hid_ref, box_ref, sc3_ref, id3_ref, w1_ref, b1_ref,
              scw1_ref, scb1_ref, wcat_ref, bsum_ref, g_ref, beta_ref,
              out_ref):
    idT = jnp.transpose(id3_ref[0])
    scT = jnp.transpose(sc3_ref[0])
    icol = jnp.concatenate(
        [idT[:, 0:1], idT[:, 1:2], idT[:, 2:3]], axis=0)[:_KP]
    scol = jnp.concatenate(
        [scT[:, 0:1], scT[:, 1:2], scT[:, 2:3]], axis=0)[:_KP]
    lanes = lax.broadcasted_iota(jnp.int32, (_KP, _N), 1)
    oh = (lanes == icol).astype(jnp.float32)
    boxg = jnp.dot(oh, box_ref[0], preferred_element_type=jnp.float32)
    h_box = _gelu_exact(jnp.dot(boxg, w1_ref[...],
                                preferred_element_type=jnp.float32)
                        + b1_ref[...])
    h_sc = _gelu_exact(scol * scw1_ref[...] + scb1_ref[...])
    hcat = jnp.concatenate([h_box, h_sc], axis=1).astype(jnp.bfloat16)
    t = jnp.dot(hcat, wcat_ref[...], preferred_element_type=jnp.float32)
    tok = hid_ref[0] + t + bsum_ref[...]
    mu = jnp.mean(tok, axis=1, keepdims=True)
    var = jnp.mean((tok - mu) ** 2, axis=1, keepdims=True)
    res = (tok - mu) * lax.rsqrt(var + 1e-5) * g_ref[...] + beta_ref[...]
    out_ref[0] = res[:_K, :]


def _run_mlp(hid, boxes, sc3, id3, box_W1, box_b1, sc_W1, sc_b1, wcat, bsum,
             ln_g, ln_b):
    blk = lambda i: (i, 0, 0)
    rep = lambda i: (0, 0)
    return pl.pallas_call(
        _mlp_body,
        grid=(_B,),
        in_specs=[
            pl.BlockSpec((1, _KP, _D), blk),
            pl.BlockSpec((1, _N, 4), blk),
            pl.BlockSpec((1, 8, 128), blk),
            pl.BlockSpec((1, 8, 128), blk),
            pl.BlockSpec((4, _D), rep),
            pl.BlockSpec((1, _D), rep),
            pl.BlockSpec((1, _D), rep),
            pl.BlockSpec((1, _D), rep),
            pl.BlockSpec((2 * _D, _D), rep),
            pl.BlockSpec((1, _D), rep),
            pl.BlockSpec((1, _D), rep),
            pl.BlockSpec((1, _D), rep),
        ],
        out_specs=pl.BlockSpec((1, _K, _D), blk),
        out_shape=jax.ShapeDtypeStruct((_B, _K, _D), jnp.float32),
    )(hid, boxes, sc3, id3, box_W1, box_b1, sc_W1, sc_b1, wcat, bsum,
      ln_g, ln_b)


def kernel(lesion_hidden, lesion_boxes, lesion_logits, box_W1, box_b1,
           box_W2, box_b2, sc_W1, sc_b1, sc_W2, sc_b2, ln_g, ln_b):
    topk_scores, topk_indices, sc3, id3, flat3 = _run_topk(lesion_logits)

    hidden_t = jnp.transpose(lesion_hidden, (1, 0, 2)).reshape(_N * _B, _D)
    hidg = _sc_gather(hidden_t, flat3.reshape(_B * 384))

    wcat = jnp.concatenate([box_W2, sc_W2], axis=0).astype(jnp.bfloat16)
    bsum = (box_b2 + sc_b2).reshape(1, _D)
    out = _run_mlp(
        hidg.reshape(_B, _KP, _D), lesion_boxes, sc3, id3,
        box_W1, box_b1.reshape(1, _D), sc_W1, sc_b1.reshape(1, _D),
        wcat, bsum, ln_g.reshape(1, _D), ln_b.reshape(1, _D))
    return (out, topk_scores, topk_indices)
